# Initial kernel scaffold; baseline (speedup 1.0000x reference)
import functools

import jax
import jax.numpy as jnp
from jax import lax
from jax.experimental import pallas as pl
from jax.experimental.pallas import tpu as pltpu
from jax.experimental.pallas import tpu_sc as plsc

DIM = 32
CHUNK = 128


@functools.lru_cache(maxsize=None)
def _make_lookup(num_emb, batch, nw, nchunks):
    mesh = plsc.VectorSubcoreMesh(core_axis_name="c", subcore_axis_name="s")
    b_per_w = nchunks * CHUNK

    @functools.partial(
        pl.kernel,
        out_type=jax.ShapeDtypeStruct((batch, DIM), jnp.float32),
        mesh=mesh,
        scratch_types=[
            pltpu.VMEM((nchunks, CHUNK), jnp.int32),
            pltpu.VMEM((CHUNK, DIM), jnp.float32),
            pltpu.SemaphoreType.DMA,
        ],
    )
    def lookup(table_hbm, idx_hbm, out_hbm, idx_v, rows_v, sem):
        wid = lax.axis_index("s") * mesh.num_cores + lax.axis_index("c")
        pltpu.sync_copy(idx_hbm.at[wid], idx_v)
        base = wid * b_per_w

        def chunk_body(j, carry):
            pltpu.async_copy(table_hbm.at[idx_v.at[j]], rows_v, sem).wait()
            pltpu.sync_copy(rows_v, out_hbm.at[pl.ds(base + j * CHUNK, CHUNK)])
            return carry

        lax.fori_loop(0, nchunks, chunk_body, 0)

    return lookup


def kernel(token_ids, embedding_matrix):
    bsz, hist = token_ids.shape
    total = bsz * hist
    info = plsc.get_sparse_core_info()
    nw = info.num_cores * info.num_subcores
    nchunks = total // (nw * CHUNK)
    assert nchunks * nw * CHUNK == total
    idx = token_ids.reshape(nw, nchunks, CHUNK).astype(jnp.int32)
    lookup = _make_lookup(embedding_matrix.shape[0], total, nw, nchunks)
    out = lookup(embedding_matrix, idx)
    return out.reshape(bsz, hist, DIM)


# speedup vs baseline: 1.0224x; 1.0224x over previous
import functools

import jax
import jax.numpy as jnp
from jax import lax
from jax.experimental import pallas as pl
from jax.experimental.pallas import tpu as pltpu
from jax.experimental.pallas import tpu_sc as plsc

DIM = 32
CHUNK = 128


@functools.lru_cache(maxsize=None)
def _make_lookup(num_emb, batch, nw, nchunks):
    mesh = plsc.VectorSubcoreMesh(core_axis_name="c", subcore_axis_name="s")
    b_per_w = nchunks * CHUNK

    @functools.partial(
        pl.kernel,
        out_type=jax.ShapeDtypeStruct((batch, DIM), jnp.float32),
        mesh=mesh,
        scratch_types=[
            pltpu.VMEM((nchunks, CHUNK), jnp.int32),
            pltpu.VMEM((CHUNK, DIM), jnp.float32),
            pltpu.SemaphoreType.DMA,
        ],
        compiler_params=pltpu.CompilerParams(use_tc_tiling_on_sc=False),
    )
    def lookup(table_hbm, idx_hbm, out_hbm, idx_v, rows_v, sem):
        wid = lax.axis_index("s") * mesh.num_cores + lax.axis_index("c")
        pltpu.sync_copy(idx_hbm.at[wid], idx_v)
        base = wid * b_per_w

        def chunk_body(j, carry):
            pltpu.async_copy(table_hbm.at[idx_v.at[j]], rows_v, sem).wait()
            pltpu.sync_copy(rows_v, out_hbm.at[pl.ds(base + j * CHUNK, CHUNK)])
            return carry

        lax.fori_loop(0, nchunks, chunk_body, 0)

    return lookup


def kernel(token_ids, embedding_matrix):
    bsz, hist = token_ids.shape
    total = bsz * hist
    info = plsc.get_sparse_core_info()
    nw = info.num_cores * info.num_subcores
    nchunks = total // (nw * CHUNK)
    assert nchunks * nw * CHUNK == total
    idx = token_ids.reshape(nw, nchunks, CHUNK).astype(jnp.int32)
    lookup = _make_lookup(embedding_matrix.shape[0], total, nw, nchunks)
    out = lookup(embedding_matrix, idx)
    return out.reshape(bsz, hist, DIM)


# speedup vs baseline: 1.1123x; 1.0879x over previous
import functools

import jax
import jax.numpy as jnp
from jax import lax
from jax.experimental import pallas as pl
from jax.experimental.pallas import tpu as pltpu
from jax.experimental.pallas import tpu_sc as plsc

DIM = 32
CHUNK = 128
K = 10


@functools.lru_cache(maxsize=None)
def _make_lookup(num_emb, batch, nw, nchunks):
    mesh = plsc.VectorSubcoreMesh(core_axis_name="c", subcore_axis_name="s")
    b_per_w = nchunks * CHUNK
    nsteps = nchunks // K
    assert nsteps * K == nchunks and nsteps % 2 == 0
    half = nsteps // 2
    step_rows = K * CHUNK

    @functools.partial(
        pl.kernel,
        out_type=jax.ShapeDtypeStruct((batch, DIM), jnp.float32),
        mesh=mesh,
        scratch_types=[
            pltpu.VMEM((nchunks, CHUNK), jnp.int32),
            pltpu.VMEM((2, step_rows, DIM), jnp.float32),
            pltpu.SemaphoreType.DMA,
            pltpu.SemaphoreType.DMA,
            pltpu.SemaphoreType.DMA,
            pltpu.SemaphoreType.DMA,
        ],
        compiler_params=pltpu.CompilerParams(use_tc_tiling_on_sc=False),
    )
    def lookup(table_hbm, idx_hbm, out_hbm, idx_v, rows_v, g0, g1, o0, o1):
        gsems = (g0, g1)
        osems = (o0, o1)
        wid = lax.axis_index("s") * mesh.num_cores + lax.axis_index("c")
        pltpu.sync_copy(idx_hbm.at[wid], idx_v)
        base = wid * b_per_w

        def fire(step, p):
            for b in range(K):
                pltpu.async_copy(
                    table_hbm.at[idx_v.at[step * K + b]],
                    rows_v.at[p].at[pl.ds(b * CHUNK, CHUNK)],
                    gsems[p],
                )

        def drain_gathers(p):
            pltpu.make_async_copy(
                out_hbm.at[pl.ds(0, step_rows)], rows_v.at[p], gsems[p]
            ).wait()

        def start_out(step, p):
            pltpu.async_copy(
                rows_v.at[p],
                out_hbm.at[pl.ds(base + step * step_rows, step_rows)],
                osems[p],
            )

        def wait_out(p):
            pltpu.make_async_copy(
                rows_v.at[p], out_hbm.at[pl.ds(0, step_rows)], osems[p]
            ).wait()

        fire(0, 0)

        @pl.loop(0, half)
        def superstep(t):
            @pl.when(t >= 1)
            def _():
                wait_out(1)

            fire(2 * t + 1, 1)
            drain_gathers(0)
            start_out(2 * t, 0)

            @pl.when(t < half - 1)
            def _():
                wait_out(0)
                fire(2 * t + 2, 0)

            drain_gathers(1)
            start_out(2 * t + 1, 1)

        wait_out(0)
        wait_out(1)

    return lookup


def kernel(token_ids, embedding_matrix):
    bsz, hist = token_ids.shape
    total = bsz * hist
    info = plsc.get_sparse_core_info()
    nw = info.num_cores * info.num_subcores
    nchunks = total // (nw * CHUNK)
    assert nchunks * nw * CHUNK == total
    idx = token_ids.reshape(nw, nchunks, CHUNK).astype(jnp.int32)
    lookup = _make_lookup(embedding_matrix.shape[0], total, nw, nchunks)
    out = lookup(embedding_matrix, idx)
    return out.reshape(bsz, hist, DIM)
